# baseline (speedup 1.0000x reference)
import functools

import jax
import jax.numpy as jnp
from jax.experimental import pallas as pl
from jax.experimental.pallas import tpu as pltpu
from jax.experimental.pallas import tpu_sc as plsc

N_RETINA = 4096
N_LGN = 1024
MU_WTS = 2.5
ETA = 0.1

_R_BLK = 512
_L_BLK = 256
_N_STEP1 = N_RETINA // _R_BLK
_N_STEP2 = N_LGN // _L_BLK

_NC, _NS, _LANES = 2, 16, 16
_NW = _NC * _NS
_ROWS_PER_TILE = N_LGN // _NW
_CHUNKS_LGN = N_LGN // _LANES
_CHUNKS_RET = N_RETINA // _LANES


def _matvecs(f_ref, nw_ref, nthr_ref, lw_ref,
             fir_out_ref, act_ref, fir_ref):
    i = pl.program_id(0)

    @pl.when(i < _N_STEP1)
    def _stage1():
        x = jnp.sum(nw_ref[...] * f_ref[...], axis=1, keepdims=True)
        xr = jnp.transpose(x, (1, 0))
        bits = (xr > nthr_ref[...]).astype(jnp.float32)
        fir_out_ref[...] = bits
        fir_ref[:, pl.ds(i * _R_BLK, _R_BLK)] = bits

    @pl.when(i >= _N_STEP1)
    def _stage2():
        j = i - _N_STEP1
        a = jnp.sum(lw_ref[...] * fir_ref[...], axis=1, keepdims=True)
        ar = jnp.transpose(a, (1, 0))
        act_ref[:, pl.ds(j * _L_BLK, _L_BLK)] = jnp.maximum(ar, 0.0)


def _sc_winner(act_hbm, thr_hbm, fir_hbm, w_hbm, w_out, thr_out,
               act_v, thr_v, tout_v, fir_v, row_v):
    c = jax.lax.axis_index("c")
    s = jax.lax.axis_index("s")
    wid = s * _NC + c
    base = wid * _ROWS_PER_TILE

    pltpu.sync_copy(w_hbm.at[pl.ds(base, _ROWS_PER_TILE)],
                    w_out.at[pl.ds(base, _ROWS_PER_TILE)])

    pltpu.sync_copy(act_hbm, act_v)
    pltpu.sync_copy(thr_hbm, thr_v)
    iota16 = jax.lax.iota(jnp.int32, _LANES)

    def _maxbody(k, mv):
        a = jnp.maximum(act_v[pl.ds(k * _LANES, _LANES)]
                        - thr_v[pl.ds(k * _LANES, _LANES)], 0.0)
        return jnp.maximum(mv, jnp.max(a))

    max_val = jax.lax.fori_loop(0, _CHUNKS_LGN, _maxbody, jnp.float32(0.0))

    def _idxbody(k, cur):
        a = jnp.maximum(act_v[pl.ds(k * _LANES, _LANES)]
                        - thr_v[pl.ds(k * _LANES, _LANES)], 0.0)
        cand = jnp.where(a == max_val, iota16 + k * _LANES, jnp.int32(N_LGN))
        return jnp.minimum(cur, jnp.min(cand))

    max_idx = jax.lax.fori_loop(0, _CHUNKS_LGN, _idxbody, jnp.int32(N_LGN))

    @pl.when(wid == 0)
    def _thr():
        def _tbody(k, carry):
            t = thr_v[pl.ds(k * _LANES, _LANES)]
            cand = iota16 + k * _LANES
            tout_v[pl.ds(k * _LANES, _LANES)] = jnp.where(
                cand == max_idx, t + 0.005 * max_val, t)
            return carry

        jax.lax.fori_loop(0, _CHUNKS_LGN, _tbody, jnp.int32(0))
        pltpu.sync_copy(tout_v, thr_out)

    owner = (base <= max_idx) & (max_idx < base + _ROWS_PER_TILE)

    @pl.when((max_val > 0.0) & owner)
    def _row():
        pltpu.sync_copy(fir_hbm, fir_v)
        pltpu.sync_copy(w_hbm.at[max_idx], row_v)
        scale = ETA * max_val

        def _rbody(k, ssum):
            r = (row_v[pl.ds(k * _LANES, _LANES)]
                 + scale * fir_v[pl.ds(k * _LANES, _LANES)])
            row_v[pl.ds(k * _LANES, _LANES)] = r
            return ssum + jnp.sum(r)

        total = jax.lax.fori_loop(0, _CHUNKS_RET, _rbody, jnp.float32(0.0))
        mean = total * (1.0 / N_RETINA)

        def _nbody(k, carry):
            row_v[pl.ds(k * _LANES, _LANES)] = (
                row_v[pl.ds(k * _LANES, _LANES)] / mean * MU_WTS)
            return carry

        jax.lax.fori_loop(0, _CHUNKS_RET, _nbody, jnp.int32(0))
        pltpu.sync_copy(row_v, w_out.at[max_idx])


def kernel(is_firing, node_weights, node_threshold, lgn_weights, lgn_threshold):
    f0 = is_firing.reshape(1, N_RETINA)
    nthr = node_threshold.reshape(1, N_RETINA)
    n_steps = _N_STEP1 + _N_STEP2

    firing, act_raw = pl.pallas_call(
        _matvecs,
        grid=(n_steps,),
        in_specs=[
            pl.BlockSpec((1, N_RETINA), lambda i: (0, 0)),
            pl.BlockSpec((_R_BLK, N_RETINA),
                         lambda i: (jnp.minimum(i, _N_STEP1 - 1), 0)),
            pl.BlockSpec((1, _R_BLK),
                         lambda i: (0, jnp.minimum(i, _N_STEP1 - 1))),
            pl.BlockSpec((_L_BLK, N_RETINA),
                         lambda i: (jnp.clip(i - _N_STEP1, 0, _N_STEP2 - 1), 0)),
        ],
        out_specs=[
            pl.BlockSpec((1, _R_BLK),
                         lambda i: (0, jnp.minimum(i, _N_STEP1 - 1))),
            pl.BlockSpec((1, N_LGN), lambda i: (0, 0)),
        ],
        out_shape=[
            jax.ShapeDtypeStruct((1, N_RETINA), jnp.float32),
            jax.ShapeDtypeStruct((1, N_LGN), jnp.float32),
        ],
        scratch_shapes=[
            pltpu.VMEM((1, N_RETINA), jnp.float32),
        ],
    )(f0, node_weights, nthr, lgn_weights)

    firing_flat = firing.reshape(N_RETINA)
    act_flat = act_raw.reshape(N_LGN)

    sc_call = functools.partial(
        pl.kernel,
        out_type=(
            jax.ShapeDtypeStruct((N_LGN, N_RETINA), jnp.float32),
            jax.ShapeDtypeStruct((N_LGN,), jnp.float32),
        ),
        mesh=plsc.VectorSubcoreMesh(
            core_axis_name="c", subcore_axis_name="s", num_cores=_NC),
        compiler_params=pltpu.CompilerParams(needs_layout_passes=False),
        scratch_types=[
            pltpu.VMEM((N_LGN,), jnp.float32),
            pltpu.VMEM((N_LGN,), jnp.float32),
            pltpu.VMEM((N_LGN,), jnp.float32),
            pltpu.VMEM((N_RETINA,), jnp.float32),
            pltpu.VMEM((N_RETINA,), jnp.float32),
        ],
    )(_sc_winner)

    new_w, new_thr = sc_call(act_flat, lgn_threshold, firing_flat, lgn_weights)

    return firing_flat, act_flat, new_w, new_thr


# speedup vs baseline: 8.0505x; 8.0505x over previous
import functools

import jax
import jax.numpy as jnp
from jax.experimental import pallas as pl
from jax.experimental.pallas import tpu as pltpu
from jax.experimental.pallas import tpu_sc as plsc

N_RETINA = 4096
N_LGN = 1024
MU_WTS = 2.5
ETA = 0.1

_R_BLK = 512
_L_BLK = 256
_N_STEP1 = N_RETINA // _R_BLK
_N_STEP2 = N_LGN // _L_BLK

_NC, _NS, _LANES = 2, 16, 16
_NW = _NC * _NS
_ROWS_PER_TILE = N_LGN // _NW
_CHUNKS_LGN = N_LGN // _LANES
_CHUNKS_RET = N_RETINA // _LANES


def _matvecs(f_ref, nw_ref, nthr_ref, lw_ref,
             fir_out_ref, act_ref, fir_ref):
    i = pl.program_id(0)

    @pl.when(i < _N_STEP1)
    def _stage1():
        x = jnp.sum(nw_ref[...] * f_ref[...], axis=1, keepdims=True)
        xr = jnp.transpose(x, (1, 0))
        bits = (xr > nthr_ref[...]).astype(jnp.float32)
        fir_out_ref[...] = bits
        fir_ref[:, pl.ds(i * _R_BLK, _R_BLK)] = bits

    @pl.when(i >= _N_STEP1)
    def _stage2():
        j = i - _N_STEP1
        a = jnp.sum(lw_ref[...] * fir_ref[...], axis=1, keepdims=True)
        ar = jnp.transpose(a, (1, 0))
        act_ref[:, pl.ds(j * _L_BLK, _L_BLK)] = jnp.maximum(ar, 0.0)


def _sc_winner(act_hbm, thr_hbm, fir_hbm, w_hbm, w_out, thr_out,
               act_v, thr_v, tout_v, fir_v, row_v, slab_v):
    c = jax.lax.axis_index("c")
    s = jax.lax.axis_index("s")
    wid = s * _NC + c
    base = wid * _ROWS_PER_TILE

    def _cbody(k, carry):
        pltpu.sync_copy(w_hbm.at[pl.ds(base + k * 8, 8)], slab_v)
        pltpu.sync_copy(slab_v, w_out.at[pl.ds(base + k * 8, 8)])
        return carry

    jax.lax.fori_loop(0, _ROWS_PER_TILE // 8, _cbody, jnp.int32(0))

    pltpu.sync_copy(act_hbm, act_v)
    pltpu.sync_copy(thr_hbm, thr_v)
    iota16 = jax.lax.iota(jnp.int32, _LANES)

    def _maxbody(k, mv):
        a = jnp.maximum(act_v[pl.ds(k * _LANES, _LANES)]
                        - thr_v[pl.ds(k * _LANES, _LANES)], 0.0)
        return jnp.maximum(mv, jnp.max(a))

    max_val = jax.lax.fori_loop(0, _CHUNKS_LGN, _maxbody, jnp.float32(0.0))

    def _idxbody(k, cur):
        a = jnp.maximum(act_v[pl.ds(k * _LANES, _LANES)]
                        - thr_v[pl.ds(k * _LANES, _LANES)], 0.0)
        cand = jnp.where(a == max_val, iota16 + k * _LANES, jnp.int32(N_LGN))
        return jnp.minimum(cur, jnp.min(cand))

    max_idx = jax.lax.fori_loop(0, _CHUNKS_LGN, _idxbody, jnp.int32(N_LGN))

    @pl.when(wid == 0)
    def _thr():
        def _tbody(k, carry):
            t = thr_v[pl.ds(k * _LANES, _LANES)]
            cand = iota16 + k * _LANES
            tout_v[pl.ds(k * _LANES, _LANES)] = jnp.where(
                cand == max_idx, t + 0.005 * max_val, t)
            return carry

        jax.lax.fori_loop(0, _CHUNKS_LGN, _tbody, jnp.int32(0))
        pltpu.sync_copy(tout_v, thr_out)

    owner = (base <= max_idx) & (max_idx < base + _ROWS_PER_TILE)

    @pl.when((max_val > 0.0) & owner)
    def _row():
        pltpu.sync_copy(fir_hbm, fir_v)
        pltpu.sync_copy(w_hbm.at[max_idx], row_v)
        scale = ETA * max_val

        def _rbody(k, ssum):
            r = (row_v[pl.ds(k * _LANES, _LANES)]
                 + scale * fir_v[pl.ds(k * _LANES, _LANES)])
            row_v[pl.ds(k * _LANES, _LANES)] = r
            return ssum + jnp.sum(r)

        total = jax.lax.fori_loop(0, _CHUNKS_RET, _rbody, jnp.float32(0.0))
        mean = total * (1.0 / N_RETINA)

        def _nbody(k, carry):
            row_v[pl.ds(k * _LANES, _LANES)] = (
                row_v[pl.ds(k * _LANES, _LANES)] / mean * MU_WTS)
            return carry

        jax.lax.fori_loop(0, _CHUNKS_RET, _nbody, jnp.int32(0))
        pltpu.sync_copy(row_v, w_out.at[max_idx])


def kernel(is_firing, node_weights, node_threshold, lgn_weights, lgn_threshold):
    f0 = is_firing.reshape(1, N_RETINA)
    nthr = node_threshold.reshape(1, N_RETINA)
    n_steps = _N_STEP1 + _N_STEP2

    firing, act_raw = pl.pallas_call(
        _matvecs,
        grid=(n_steps,),
        in_specs=[
            pl.BlockSpec((1, N_RETINA), lambda i: (0, 0)),
            pl.BlockSpec((_R_BLK, N_RETINA),
                         lambda i: (jnp.minimum(i, _N_STEP1 - 1), 0)),
            pl.BlockSpec((1, _R_BLK),
                         lambda i: (0, jnp.minimum(i, _N_STEP1 - 1))),
            pl.BlockSpec((_L_BLK, N_RETINA),
                         lambda i: (jnp.clip(i - _N_STEP1, 0, _N_STEP2 - 1), 0)),
        ],
        out_specs=[
            pl.BlockSpec((1, _R_BLK),
                         lambda i: (0, jnp.minimum(i, _N_STEP1 - 1))),
            pl.BlockSpec((1, N_LGN), lambda i: (0, 0)),
        ],
        out_shape=[
            jax.ShapeDtypeStruct((1, N_RETINA), jnp.float32),
            jax.ShapeDtypeStruct((1, N_LGN), jnp.float32),
        ],
        scratch_shapes=[
            pltpu.VMEM((1, N_RETINA), jnp.float32),
        ],
    )(f0, node_weights, nthr, lgn_weights)

    firing_flat = firing.reshape(N_RETINA)
    act_flat = act_raw.reshape(N_LGN)

    sc_call = functools.partial(
        pl.kernel,
        out_type=(
            jax.ShapeDtypeStruct((N_LGN, N_RETINA), jnp.float32),
            jax.ShapeDtypeStruct((N_LGN,), jnp.float32),
        ),
        mesh=plsc.VectorSubcoreMesh(
            core_axis_name="c", subcore_axis_name="s", num_cores=_NC),
        compiler_params=pltpu.CompilerParams(needs_layout_passes=False),
        scratch_types=[
            pltpu.VMEM((N_LGN,), jnp.float32),
            pltpu.VMEM((N_LGN,), jnp.float32),
            pltpu.VMEM((N_LGN,), jnp.float32),
            pltpu.VMEM((N_RETINA,), jnp.float32),
            pltpu.VMEM((N_RETINA,), jnp.float32),
            pltpu.VMEM((8, N_RETINA), jnp.float32),
        ],
    )(_sc_winner)

    new_w, new_thr = sc_call(act_flat, lgn_threshold, firing_flat, lgn_weights)

    return firing_flat, act_flat, new_w, new_thr


# speedup vs baseline: 9.5528x; 1.1866x over previous
import functools

import jax
import jax.numpy as jnp
from jax.experimental import pallas as pl
from jax.experimental.pallas import tpu as pltpu
from jax.experimental.pallas import tpu_sc as plsc

N_RETINA = 4096
N_LGN = 1024
MU_WTS = 2.5
ETA = 0.1

_R_BLK = 512
_L_BLK = 256
_N_STEP1 = N_RETINA // _R_BLK
_N_STEP2 = N_LGN // _L_BLK

_NC, _NS, _LANES = 2, 16, 16
_UNROLL = 4
_N_RCHUNK = N_RETINA // (_LANES * _UNROLL)


def _mega(f_ref, nw_ref, nthr_ref, lw_ref, lthr_ref,
          fir_out_ref, act_ref, w_out_ref, thr_out_ref, sf_ref, si_ref,
          fir_ref, sem):
    i = pl.program_id(0)

    @pl.when(i < _N_STEP1)
    def _stage1():
        x = jnp.sum(nw_ref[...] * f_ref[...], axis=1, keepdims=True)
        xr = jnp.transpose(x, (1, 0))
        bits = (xr > nthr_ref[...]).astype(jnp.float32)
        fir_out_ref[...] = bits
        fir_ref[:, pl.ds(i * _R_BLK, _R_BLK)] = bits

    @pl.when((i >= _N_STEP1) & (i < _N_STEP1 + _N_STEP2))
    def _stage2():
        j = i - _N_STEP1
        cp = pltpu.make_async_copy(
            lw_ref, w_out_ref.at[pl.ds(j * _L_BLK, _L_BLK)], sem)
        cp.start()
        a = jnp.sum(lw_ref[...] * fir_ref[...], axis=1, keepdims=True)
        ar = jnp.transpose(a, (1, 0))
        act_ref[:, pl.ds(j * _L_BLK, _L_BLK)] = jnp.maximum(ar, 0.0)
        cp.wait()

    @pl.when(i == _N_STEP1 + _N_STEP2)
    def _stage3():
        act = jnp.maximum(act_ref[...] - lthr_ref[...], 0.0)
        max_val = jnp.max(act)
        idx = jax.lax.broadcasted_iota(jnp.int32, (1, N_LGN), 1)
        max_idx = jnp.min(jnp.where(act == max_val, idx, jnp.int32(N_LGN)))
        thr_out_ref[...] = lthr_ref[...] + jnp.where(
            idx == max_idx, 0.005 * max_val, 0.0)
        sf_ref[...] = jnp.full((1, 128), max_val, jnp.float32)
        si_ref[...] = jnp.full((1, 128), max_idx, jnp.int32)


def _sc_scatter(fir_hbm, scalf_hbm, scali_hbm, w_ref,
                fir_v, row_v, sf_v, si_v):
    c = jax.lax.axis_index("c")
    s = jax.lax.axis_index("s")
    wid = s * _NC + c

    @pl.when(wid == 0)
    def _tile0():
        pltpu.sync_copy(scalf_hbm, sf_v)
        pltpu.sync_copy(scali_hbm, si_v)
        max_val = jnp.max(sf_v[pl.ds(0, _LANES)])
        max_idx = jnp.max(si_v[pl.ds(0, _LANES)])

        @pl.when(max_val > 0.0)
        def _update():
            pltpu.sync_copy(fir_hbm, fir_v)
            pltpu.sync_copy(w_ref.at[max_idx], row_v)
            scale = ETA * max_val

            def _rbody(k, ssum):
                acc = ssum
                for u in range(_UNROLL):
                    off = (k * _UNROLL + u) * _LANES
                    r = (row_v[pl.ds(off, _LANES)]
                         + scale * fir_v[pl.ds(off, _LANES)])
                    row_v[pl.ds(off, _LANES)] = r
                    acc = acc + jnp.sum(r)
                return acc

            total = jax.lax.fori_loop(0, _N_RCHUNK, _rbody, jnp.float32(0.0))
            mean_v = (total * (1.0 / N_RETINA)) * jnp.ones((_LANES,),
                                                           jnp.float32)

            def _nbody(k, carry):
                for u in range(_UNROLL):
                    off = (k * _UNROLL + u) * _LANES
                    row_v[pl.ds(off, _LANES)] = (
                        row_v[pl.ds(off, _LANES)] / mean_v * MU_WTS)
                return carry

            jax.lax.fori_loop(0, _N_RCHUNK, _nbody, jnp.int32(0))
            pltpu.sync_copy(row_v, w_ref.at[max_idx])


def kernel(is_firing, node_weights, node_threshold, lgn_weights, lgn_threshold):
    f0 = is_firing.reshape(1, N_RETINA)
    nthr = node_threshold.reshape(1, N_RETINA)
    lthr = lgn_threshold.reshape(1, N_LGN)
    n_steps = _N_STEP1 + _N_STEP2 + 1

    firing, act_raw, w_copy, new_thr, scal_f, scal_i = pl.pallas_call(
        _mega,
        grid=(n_steps,),
        in_specs=[
            pl.BlockSpec((1, N_RETINA), lambda i: (0, 0)),
            pl.BlockSpec((_R_BLK, N_RETINA),
                         lambda i: (jnp.minimum(i, _N_STEP1 - 1), 0)),
            pl.BlockSpec((1, _R_BLK),
                         lambda i: (0, jnp.minimum(i, _N_STEP1 - 1))),
            pl.BlockSpec((_L_BLK, N_RETINA),
                         lambda i: (jnp.clip(i - _N_STEP1, 0, _N_STEP2 - 1), 0)),
            pl.BlockSpec((1, N_LGN), lambda i: (0, 0)),
        ],
        out_specs=[
            pl.BlockSpec((1, _R_BLK),
                         lambda i: (0, jnp.minimum(i, _N_STEP1 - 1))),
            pl.BlockSpec((1, N_LGN), lambda i: (0, 0)),
            pl.BlockSpec(memory_space=pl.ANY),
            pl.BlockSpec((1, N_LGN), lambda i: (0, 0)),
            pl.BlockSpec((1, 128), lambda i: (0, 0)),
            pl.BlockSpec((1, 128), lambda i: (0, 0)),
        ],
        out_shape=[
            jax.ShapeDtypeStruct((1, N_RETINA), jnp.float32),
            jax.ShapeDtypeStruct((1, N_LGN), jnp.float32),
            jax.ShapeDtypeStruct((N_LGN, N_RETINA), jnp.float32),
            jax.ShapeDtypeStruct((1, N_LGN), jnp.float32),
            jax.ShapeDtypeStruct((1, 128), jnp.float32),
            jax.ShapeDtypeStruct((1, 128), jnp.int32),
        ],
        scratch_shapes=[
            pltpu.VMEM((1, N_RETINA), jnp.float32),
            pltpu.SemaphoreType.DMA,
        ],
    )(f0, node_weights, nthr, lgn_weights, lthr)

    sc_scatter = functools.partial(
        pl.kernel,
        out_type=(),
        mesh=plsc.VectorSubcoreMesh(
            core_axis_name="c", subcore_axis_name="s", num_cores=_NC),
        compiler_params=pltpu.CompilerParams(needs_layout_passes=False),
        scratch_types=[
            pltpu.VMEM((N_RETINA,), jnp.float32),
            pltpu.VMEM((N_RETINA,), jnp.float32),
            pltpu.VMEM((128,), jnp.float32),
            pltpu.VMEM((128,), jnp.int32),
        ],
    )(_sc_scatter)

    w_ref = jax.new_ref(w_copy)
    sc_scatter(firing.reshape(N_RETINA), scal_f.reshape(128),
               scal_i.reshape(128), w_ref)
    new_w = w_ref[...]

    return (firing.reshape(N_RETINA), act_raw.reshape(N_LGN),
            new_w, new_thr.reshape(N_LGN))


# speedup vs baseline: 9.5736x; 1.0022x over previous
import functools

import jax
import jax.numpy as jnp
from jax.experimental import pallas as pl
from jax.experimental.pallas import tpu as pltpu
from jax.experimental.pallas import tpu_sc as plsc

N_RETINA = 4096
N_LGN = 1024
MU_WTS = 2.5
ETA = 0.1

_R_BLK = 512
_L_BLK = 256
_N_STEP1 = N_RETINA // _R_BLK
_N_STEP2 = N_LGN // _L_BLK

_NC, _NS, _LANES = 2, 16, 16
_UNROLL = 4
_N_RCHUNK = N_RETINA // (_LANES * _UNROLL)


def _mega(f_ref, nw_ref, nthr_ref, lw_ref, lthr_ref,
          fir_out_ref, act_ref, w_out_ref, thr_out_ref, sf_ref, si_ref,
          fir_ref, sem):
    i = pl.program_id(0)

    @pl.when(i < _N_STEP1)
    def _stage1():
        x = jnp.sum(nw_ref[...] * f_ref[...], axis=1, keepdims=True)
        xr = jnp.transpose(x, (1, 0))
        bits = (xr > nthr_ref[...]).astype(jnp.float32)
        fir_out_ref[...] = bits
        fir_ref[:, pl.ds(i * _R_BLK, _R_BLK)] = bits

    @pl.when((i >= _N_STEP1) & (i < _N_STEP1 + _N_STEP2))
    def _stage2():
        j = i - _N_STEP1
        cp = pltpu.make_async_copy(
            lw_ref, w_out_ref.at[pl.ds(j * _L_BLK, _L_BLK)], sem)
        cp.start()
        a = jnp.sum(lw_ref[...] * fir_ref[...], axis=1, keepdims=True)
        ar = jnp.transpose(a, (1, 0))
        act_ref[:, pl.ds(j * _L_BLK, _L_BLK)] = jnp.maximum(ar, 0.0)
        cp.wait()

    @pl.when(i == _N_STEP1 + _N_STEP2)
    def _stage3():
        act = jnp.maximum(act_ref[...] - lthr_ref[...], 0.0)
        max_val = jnp.max(act)
        idx = jax.lax.broadcasted_iota(jnp.int32, (1, N_LGN), 1)
        max_idx = jnp.min(jnp.where(act == max_val, idx, jnp.int32(N_LGN)))
        thr_out_ref[...] = lthr_ref[...] + jnp.where(
            idx == max_idx, 0.005 * max_val, 0.0)
        sf_ref[...] = jnp.full((1, 128), max_val, jnp.float32)
        si_ref[...] = jnp.full((1, 128), max_idx, jnp.int32)


def _sc_scatter(fir_hbm, scalf_hbm, scali_hbm, w_ref,
                fir_v, row_v, sf_v, si_v):
    c = jax.lax.axis_index("c")
    s = jax.lax.axis_index("s")
    wid = s * _NC + c

    @pl.when(wid == 0)
    def _tile0():
        pltpu.sync_copy(scalf_hbm, sf_v)
        pltpu.sync_copy(scali_hbm, si_v)
        max_val = jnp.max(sf_v[pl.ds(0, _LANES)])
        max_idx = jnp.max(si_v[pl.ds(0, _LANES)])

        @pl.when(max_val > 0.0)
        def _update():
            pltpu.sync_copy(fir_hbm, fir_v)
            pltpu.sync_copy(w_ref.at[max_idx], row_v)
            scale = ETA * max_val

            def _rbody(k, ssum):
                acc = ssum
                for u in range(_UNROLL):
                    off = (k * _UNROLL + u) * _LANES
                    r = (row_v[pl.ds(off, _LANES)]
                         + scale * fir_v[pl.ds(off, _LANES)])
                    row_v[pl.ds(off, _LANES)] = r
                    acc = acc + jnp.sum(r)
                return acc

            total = jax.lax.fori_loop(0, _N_RCHUNK, _rbody, jnp.float32(0.0))
            mean_v = (total * (1.0 / N_RETINA)) * jnp.ones((_LANES,),
                                                           jnp.float32)

            def _nbody(k, carry):
                for u in range(_UNROLL):
                    off = (k * _UNROLL + u) * _LANES
                    row_v[pl.ds(off, _LANES)] = (
                        row_v[pl.ds(off, _LANES)] / mean_v * MU_WTS)
                return carry

            jax.lax.fori_loop(0, _N_RCHUNK, _nbody, jnp.int32(0))
            pltpu.sync_copy(row_v, w_ref.at[max_idx])


def kernel(is_firing, node_weights, node_threshold, lgn_weights, lgn_threshold):
    f0 = is_firing.reshape(1, N_RETINA)
    nthr = node_threshold.reshape(1, N_RETINA)
    lthr = lgn_threshold.reshape(1, N_LGN)
    n_steps = _N_STEP1 + _N_STEP2 + 1

    firing, act_raw, w_copy, new_thr, scal_f, scal_i = pl.pallas_call(
        _mega,
        grid=(n_steps,),
        in_specs=[
            pl.BlockSpec((1, N_RETINA), lambda i: (0, 0)),
            pl.BlockSpec((_R_BLK, N_RETINA),
                         lambda i: (jnp.minimum(i, _N_STEP1 - 1), 0)),
            pl.BlockSpec((1, _R_BLK),
                         lambda i: (0, jnp.minimum(i, _N_STEP1 - 1))),
            pl.BlockSpec((_L_BLK, N_RETINA),
                         lambda i: (jnp.clip(i - _N_STEP1, 0, _N_STEP2 - 1), 0)),
            pl.BlockSpec((1, N_LGN), lambda i: (0, 0)),
        ],
        out_specs=[
            pl.BlockSpec((1, _R_BLK),
                         lambda i: (0, jnp.minimum(i, _N_STEP1 - 1))),
            pl.BlockSpec((1, N_LGN), lambda i: (0, 0)),
            pl.BlockSpec(memory_space=pl.ANY),
            pl.BlockSpec((1, N_LGN), lambda i: (0, 0)),
            pl.BlockSpec((1, 128), lambda i: (0, 0)),
            pl.BlockSpec((1, 128), lambda i: (0, 0)),
        ],
        out_shape=[
            jax.ShapeDtypeStruct((1, N_RETINA), jnp.float32),
            jax.ShapeDtypeStruct((1, N_LGN), jnp.float32),
            jax.ShapeDtypeStruct((N_LGN, N_RETINA), jnp.float32),
            jax.ShapeDtypeStruct((1, N_LGN), jnp.float32),
            jax.ShapeDtypeStruct((1, 128), jnp.float32),
            jax.ShapeDtypeStruct((1, 128), jnp.int32),
        ],
        scratch_shapes=[
            pltpu.VMEM((1, N_RETINA), jnp.float32),
            pltpu.SemaphoreType.DMA,
        ],
    )(f0, node_weights, nthr, lgn_weights, lthr)

    sc_scatter = functools.partial(
        pl.kernel,
        out_type=(),
        mesh=plsc.VectorSubcoreMesh(
            core_axis_name="c", subcore_axis_name="s", num_cores=_NC),
        compiler_params=pltpu.CompilerParams(needs_layout_passes=False),
        scratch_types=[
            pltpu.VMEM((N_RETINA,), jnp.float32),
            pltpu.VMEM((N_RETINA,), jnp.float32),
            pltpu.VMEM((128,), jnp.float32),
            pltpu.VMEM((128,), jnp.int32),
        ],
    )(_sc_scatter)

    w_ref = jax.new_ref(w_copy)
    sc_scatter(firing.reshape(N_RETINA), scal_f.reshape(128),
               scal_i.reshape(128), w_ref)
    new_w = jax.freeze(w_ref)

    return (firing.reshape(N_RETINA), act_raw.reshape(N_LGN),
            new_w, new_thr.reshape(N_LGN))


# speedup vs baseline: 9.8200x; 1.0257x over previous
import functools

import jax
import jax.numpy as jnp
from jax.experimental import pallas as pl
from jax.experimental.pallas import tpu as pltpu
from jax.experimental.pallas import tpu_sc as plsc

N_RETINA = 4096
N_LGN = 1024
MU_WTS = 2.5
ETA = 0.1

_R_BLK = 512
_L_BLK = 256
_N_STEP1 = N_RETINA // _R_BLK
_N_STEP2 = N_LGN // _L_BLK

_NC, _NS, _LANES = 2, 16, 16
_UNROLL = 4
_N_RCHUNK = N_RETINA // (_LANES * _UNROLL)


def _mega(f_ref, nw_ref, nthr_ref, lw_ref, lthr_ref,
          fir_out_ref, act_ref, w_out_ref, thr_out_ref, sf_ref, si_ref,
          fir_ref, sem):
    i = pl.program_id(0)

    @pl.when(i < _N_STEP1)
    def _stage1():
        x = jnp.sum(nw_ref[...] * f_ref[...], axis=1, keepdims=True)
        xr = jnp.transpose(x, (1, 0))
        bits = (xr > nthr_ref[...]).astype(jnp.float32)
        fir_out_ref[...] = bits
        fir_ref[:, pl.ds(i * _R_BLK, _R_BLK)] = bits

    @pl.when((i >= _N_STEP1) & (i < _N_STEP1 + _N_STEP2))
    def _stage2():
        j = i - _N_STEP1
        cp = pltpu.make_async_copy(
            lw_ref, w_out_ref.at[pl.ds(j * _L_BLK, _L_BLK)], sem)
        cp.start()
        a = jnp.sum(lw_ref[...] * fir_ref[...], axis=1, keepdims=True)
        ar = jnp.transpose(a, (1, 0))
        act_ref[:, pl.ds(j * _L_BLK, _L_BLK)] = jnp.maximum(ar, 0.0)
        cp.wait()

    @pl.when(i == _N_STEP1 + _N_STEP2)
    def _stage3():
        act = jnp.maximum(act_ref[...] - lthr_ref[...], 0.0)
        max_val = jnp.max(act)
        idx = jax.lax.broadcasted_iota(jnp.int32, (1, N_LGN), 1)
        max_idx = jnp.min(jnp.where(act == max_val, idx, jnp.int32(N_LGN)))
        thr_out_ref[...] = lthr_ref[...] + jnp.where(
            idx == max_idx, 0.005 * max_val, 0.0)
        sf_ref[...] = jnp.full((1, 128), max_val, jnp.float32)
        si_ref[...] = jnp.full((1, 128), max_idx, jnp.int32)


def _sc_scatter(fir_hbm, scalf_hbm, scali_hbm, w_ref,
                fir_v, row_v, sf_v, si_v):
    c = jax.lax.axis_index("c")
    s = jax.lax.axis_index("s")
    wid = s * _NC + c

    @pl.when(wid == 0)
    def _tile0():
        pltpu.sync_copy(scalf_hbm, sf_v)
        pltpu.sync_copy(scali_hbm, si_v)
        max_val = jnp.max(sf_v[pl.ds(0, _LANES)])
        max_idx = jnp.max(si_v[pl.ds(0, _LANES)])

        @pl.when(max_val > 0.0)
        def _update():
            pltpu.sync_copy(fir_hbm, fir_v)
            pltpu.sync_copy(w_ref.at[max_idx], row_v)
            scale = ETA * max_val

            def _rbody(k, ssum):
                acc = ssum
                for u in range(_UNROLL):
                    off = (k * _UNROLL + u) * _LANES
                    r = (row_v[pl.ds(off, _LANES)]
                         + scale * fir_v[pl.ds(off, _LANES)])
                    row_v[pl.ds(off, _LANES)] = r
                    acc = acc + jnp.sum(r)
                return acc

            total = jax.lax.fori_loop(0, _N_RCHUNK, _rbody, jnp.float32(0.0))
            mean_v = (total * (1.0 / N_RETINA)) * jnp.ones((_LANES,),
                                                           jnp.float32)

            def _nbody(k, carry):
                for u in range(_UNROLL):
                    off = (k * _UNROLL + u) * _LANES
                    row_v[pl.ds(off, _LANES)] = (
                        row_v[pl.ds(off, _LANES)] / mean_v * MU_WTS)
                return carry

            jax.lax.fori_loop(0, _N_RCHUNK, _nbody, jnp.int32(0))
            pltpu.sync_copy(row_v, w_ref.at[max_idx])


def kernel(is_firing, node_weights, node_threshold, lgn_weights, lgn_threshold):
    f0 = is_firing.reshape(1, N_RETINA)
    nthr = node_threshold.reshape(1, N_RETINA)
    lthr = lgn_threshold.reshape(1, N_LGN)
    n_steps = _N_STEP1 + _N_STEP2 + 1

    firing, act_raw, w_copy, new_thr, scal_f, scal_i = pl.pallas_call(
        _mega,
        grid=(n_steps,),
        in_specs=[
            pl.BlockSpec((1, N_RETINA), lambda i: (0, 0)),
            pl.BlockSpec((_R_BLK, N_RETINA),
                         lambda i: (jnp.minimum(i, _N_STEP1 - 1), 0)),
            pl.BlockSpec((1, _R_BLK),
                         lambda i: (0, jnp.minimum(i, _N_STEP1 - 1))),
            pl.BlockSpec((_L_BLK, N_RETINA),
                         lambda i: (jnp.clip(i - _N_STEP1, 0, _N_STEP2 - 1), 0)),
            pl.BlockSpec((1, N_LGN), lambda i: (0, 0)),
        ],
        out_specs=[
            pl.BlockSpec((1, _R_BLK),
                         lambda i: (0, jnp.minimum(i, _N_STEP1 - 1))),
            pl.BlockSpec((1, N_LGN), lambda i: (0, 0)),
            pl.BlockSpec(memory_space=pl.ANY),
            pl.BlockSpec((1, N_LGN), lambda i: (0, 0)),
            pl.BlockSpec((1, 128), lambda i: (0, 0)),
            pl.BlockSpec((1, 128), lambda i: (0, 0)),
        ],
        out_shape=[
            jax.ShapeDtypeStruct((1, N_RETINA), jnp.float32),
            jax.ShapeDtypeStruct((1, N_LGN), jnp.float32),
            jax.ShapeDtypeStruct((N_LGN, N_RETINA), jnp.float32),
            jax.ShapeDtypeStruct((1, N_LGN), jnp.float32),
            jax.ShapeDtypeStruct((1, 128), jnp.float32),
            jax.ShapeDtypeStruct((1, 128), jnp.int32),
        ],
        scratch_shapes=[
            pltpu.VMEM((1, N_RETINA), jnp.float32),
            pltpu.SemaphoreType.DMA,
        ],
    )(f0, node_weights, nthr, lgn_weights, lthr)

    sc_scatter = functools.partial(
        pl.kernel,
        out_type=(),
        mesh=plsc.VectorSubcoreMesh(
            core_axis_name="c", subcore_axis_name="s", num_cores=1),
        compiler_params=pltpu.CompilerParams(
            needs_layout_passes=False, skip_device_barrier=True),
        scratch_types=[
            pltpu.VMEM((N_RETINA,), jnp.float32),
            pltpu.VMEM((N_RETINA,), jnp.float32),
            pltpu.VMEM((128,), jnp.float32),
            pltpu.VMEM((128,), jnp.int32),
        ],
    )(_sc_scatter)

    w_ref = jax.new_ref(w_copy)
    sc_scatter(firing.reshape(N_RETINA), scal_f.reshape(128),
               scal_i.reshape(128), w_ref)
    new_w = jax.freeze(w_ref)

    return (firing.reshape(N_RETINA), act_raw.reshape(N_LGN),
            new_w, new_thr.reshape(N_LGN))
